# baseline (speedup 1.0000x reference)
import functools

import jax
import jax.numpy as jnp
from jax import lax
from jax.experimental import pallas as pl
from jax.experimental.pallas import tpu as pltpu
from jax.experimental.pallas import tpu_sc as plsc

N = 4096
NUM_CORES = 2
NUM_SUBCORES = 16
NUM_WORKERS = NUM_CORES * NUM_SUBCORES
ROWS_PER_WORKER = N // NUM_WORKERS
CHUNK = 16
STEPS = ROWS_PER_WORKER // CHUNK
LANES = 16


def _sc_body(perm_hbm, out_hbm, idx_v, buf):
    c = lax.axis_index("c")
    s = lax.axis_index("s")
    wid = s * NUM_CORES + c
    base = wid * ROWS_PER_WORKER

    pltpu.sync_copy(perm_hbm.at[pl.ds(base, ROWS_PER_WORKER)], idx_v)

    zeros = jnp.zeros((LANES,), jnp.float32)
    ones = jnp.ones((LANES,), jnp.float32)
    rows = lax.iota(jnp.int32, LANES)

    def _zero_cols(j, _):
        for r in range(CHUNK):
            buf[r, pl.ds(j * LANES, LANES)] = zeros
        return 0

    lax.fori_loop(0, N // LANES, _zero_cols, 0, unroll=4)

    def _step(st, _):
        cols = idx_v[pl.ds(st * CHUNK, CHUNK)]
        plsc.store_scatter(buf, [rows, cols], ones)
        pltpu.sync_copy(buf, out_hbm.at[pl.ds(base + st * CHUNK, CHUNK)])
        plsc.store_scatter(buf, [rows, cols], zeros)
        return 0

    lax.fori_loop(0, STEPS, _step, 0)


@functools.partial(jax.jit, static_argnums=())
def _sc_build(perm):
    mesh = plsc.VectorSubcoreMesh(
        core_axis_name="c", subcore_axis_name="s",
        num_cores=NUM_CORES, num_subcores=NUM_SUBCORES,
    )
    return pl.kernel(
        _sc_body,
        out_type=jax.ShapeDtypeStruct((N, N), jnp.float32),
        mesh=mesh,
        scratch_types=[
            pltpu.VMEM((ROWS_PER_WORKER,), jnp.int32),
            pltpu.VMEM((CHUNK, N), jnp.float32),
        ],
        compiler_params=pltpu.CompilerParams(needs_layout_passes=False),
    )(perm)


def kernel(perm):
    return _sc_build(perm.astype(jnp.int32))
